# initial kernel scaffold (unmeasured)
import jax
import jax.numpy as jnp
from jax import lax
from jax.experimental import pallas as pl
from jax.experimental.pallas import tpu as pltpu


def kernel(
    t,
):
    def body(*refs):
        pass

    out_shape = jax.ShapeDtypeStruct(..., jnp.float32)
    return pl.pallas_call(body, out_shape=out_shape)(...)


# baseline (device time: 83875 ns/iter reference)
import jax
import jax.numpy as jnp
from jax import lax
from jax.experimental import pallas as pl
from jax.experimental.pallas import tpu as pltpu

N_DEV = 4


def kernel(t):
    m, n = t.shape
    half = m // 2
    quart = m // 4

    def body(t_ref, out_ref, tb_ref, recv1_ref, acc_ref, recv2_ref,
             send_sems, recv_sems):
        my = lax.axis_index("i")
        ybit = (my ^ (my >> 1)) & 1
        xbit = my >> 1
        yp = my ^ 1
        xp = 3 - my

        barrier_sem = pltpu.get_barrier_semaphore()
        for nbr in (yp, xp):
            pl.semaphore_signal(
                barrier_sem, inc=1,
                device_id=(nbr,), device_id_type=pl.DeviceIdType.MESH,
            )
        pl.semaphore_wait(barrier_sem, 2)

        tb_ref[:, :] = t_ref[:, :].astype(jnp.bfloat16)

        rdma1 = pltpu.make_async_remote_copy(
            src_ref=tb_ref.at[pl.ds((1 - ybit) * half, half), :],
            dst_ref=recv1_ref,
            send_sem=send_sems.at[0],
            recv_sem=recv_sems.at[0],
            device_id=(yp,),
            device_id_type=pl.DeviceIdType.MESH,
        )
        rdma1.start()
        rdma1.wait()
        acc_ref[:, :] = (
            tb_ref[pl.ds(ybit * half, half), :].astype(jnp.float32)
            + recv1_ref[:, :].astype(jnp.float32)
        ).astype(jnp.bfloat16)

        rdma2 = pltpu.make_async_remote_copy(
            src_ref=acc_ref.at[pl.ds((1 - xbit) * quart, quart), :],
            dst_ref=recv2_ref,
            send_sem=send_sems.at[1],
            recv_sem=recv_sems.at[1],
            device_id=(xp,),
            device_id_type=pl.DeviceIdType.MESH,
        )
        rdma2.start()
        rdma2.wait()

        s = (
            acc_ref[pl.ds(xbit * quart, quart), :].astype(jnp.float32)
            + recv2_ref[:, :].astype(jnp.float32)
        )
        r = jnp.maximum(s, 0.0)
        fq = jnp.tanh(s) * s * s + r * r * r
        q0 = ybit * half + xbit * quart
        out_ref[pl.ds(q0, quart), :] = fq.astype(jnp.bfloat16)

        rdma3 = pltpu.make_async_remote_copy(
            src_ref=out_ref.at[pl.ds(q0, quart), :],
            dst_ref=out_ref.at[pl.ds(q0, quart), :],
            send_sem=send_sems.at[2],
            recv_sem=recv_sems.at[2],
            device_id=(xp,),
            device_id_type=pl.DeviceIdType.MESH,
        )
        rdma3.start()
        rdma3.wait()

        rdma4 = pltpu.make_async_remote_copy(
            src_ref=out_ref.at[pl.ds(ybit * half, half), :],
            dst_ref=out_ref.at[pl.ds(ybit * half, half), :],
            send_sem=send_sems.at[3],
            recv_sem=recv_sems.at[3],
            device_id=(yp,),
            device_id_type=pl.DeviceIdType.MESH,
        )
        rdma4.start()
        rdma4.wait()

    return pl.pallas_call(
        body,
        out_shape=jax.ShapeDtypeStruct((m, n), jnp.bfloat16),
        in_specs=[pl.BlockSpec(memory_space=pltpu.VMEM)],
        out_specs=pl.BlockSpec(memory_space=pltpu.VMEM),
        scratch_shapes=[
            pltpu.VMEM((m, n), jnp.bfloat16),
            pltpu.VMEM((half, n), jnp.bfloat16),
            pltpu.VMEM((half, n), jnp.bfloat16),
            pltpu.VMEM((quart, n), jnp.bfloat16),
            pltpu.SemaphoreType.DMA((4,)),
            pltpu.SemaphoreType.DMA((4,)),
        ],
        compiler_params=pltpu.CompilerParams(collective_id=0),
    )(t)


# device time: 49613 ns/iter; 1.6906x vs baseline; 1.6906x over previous
import jax
import jax.numpy as jnp
from jax import lax
from jax.experimental import pallas as pl
from jax.experimental.pallas import tpu as pltpu

N_DEV = 4


def kernel(t):
    m, n = t.shape
    half = m // 2
    quart = m // 4
    nh = n // 2

    def body(t_ref, out_ref, sendA_ref, sendB_ref, recv1A_ref, recv1B_ref,
             accA_ref, accB_ref, recv2A_ref, recv2B_ref, send_sems, recv_sems):
        my = lax.axis_index("i")
        ybit = (my ^ (my >> 1)) & 1
        xbit = my >> 1
        yp = my ^ 1
        xp = 3 - my

        def rdma(src, dst, slot, dev):
            return pltpu.make_async_remote_copy(
                src_ref=src, dst_ref=dst,
                send_sem=send_sems.at[slot], recv_sem=recv_sems.at[slot],
                device_id=(dev,), device_id_type=pl.DeviceIdType.MESH,
            )

        def f(s):
            r = jnp.maximum(s, 0.0)
            return jnp.tanh(s) * s * s + r * r * r

        barrier_sem = pltpu.get_barrier_semaphore()
        for nbr in (yp, xp):
            pl.semaphore_signal(
                barrier_sem, inc=1,
                device_id=(nbr,), device_id_type=pl.DeviceIdType.MESH,
            )
        pl.semaphore_wait(barrier_sem, 2)

        sendA_ref[:, :] = t_ref[
            pl.ds((1 - ybit) * half, half), pl.ds(0, nh)
        ].astype(jnp.bfloat16)
        r1A = rdma(sendA_ref, recv1A_ref, 0, yp)
        r1A.start()
        sendB_ref[:, :] = t_ref[
            pl.ds((1 - xbit) * half, half), pl.ds(nh, nh)
        ].astype(jnp.bfloat16)
        r1B = rdma(sendB_ref, recv1B_ref, 1, xp)
        r1B.start()

        r1A.wait()
        accA_ref[:, :] = (
            t_ref[pl.ds(ybit * half, half), pl.ds(0, nh)]
            + recv1A_ref[:, :].astype(jnp.float32)
        ).astype(jnp.bfloat16)
        r2A = rdma(accA_ref.at[pl.ds((1 - xbit) * quart, quart), :],
                   recv2A_ref, 2, xp)
        r2A.start()

        r1B.wait()
        accB_ref[:, :] = (
            t_ref[pl.ds(xbit * half, half), pl.ds(nh, nh)]
            + recv1B_ref[:, :].astype(jnp.float32)
        ).astype(jnp.bfloat16)
        r2B = rdma(accB_ref.at[pl.ds((1 - ybit) * quart, quart), :],
                   recv2B_ref, 3, yp)
        r2B.start()

        r2A.wait()
        sA = (
            accA_ref[pl.ds(xbit * quart, quart), :].astype(jnp.float32)
            + recv2A_ref[:, :].astype(jnp.float32)
        )
        qA = ybit * half + xbit * quart
        out_ref[pl.ds(qA, quart), pl.ds(0, nh)] = f(sA).astype(jnp.bfloat16)
        r3A = rdma(out_ref.at[pl.ds(qA, quart), pl.ds(0, nh)],
                   out_ref.at[pl.ds(qA, quart), pl.ds(0, nh)], 4, xp)
        r3A.start()

        r2B.wait()
        sB = (
            accB_ref[pl.ds(ybit * quart, quart), :].astype(jnp.float32)
            + recv2B_ref[:, :].astype(jnp.float32)
        )
        qB = xbit * half + ybit * quart
        out_ref[pl.ds(qB, quart), pl.ds(nh, nh)] = f(sB).astype(jnp.bfloat16)
        r3B = rdma(out_ref.at[pl.ds(qB, quart), pl.ds(nh, nh)],
                   out_ref.at[pl.ds(qB, quart), pl.ds(nh, nh)], 5, yp)
        r3B.start()

        r3A.wait()
        r4A = rdma(out_ref.at[pl.ds(ybit * half, half), pl.ds(0, nh)],
                   out_ref.at[pl.ds(ybit * half, half), pl.ds(0, nh)], 6, yp)
        r4A.start()
        r3B.wait()
        r4B = rdma(out_ref.at[pl.ds(xbit * half, half), pl.ds(nh, nh)],
                   out_ref.at[pl.ds(xbit * half, half), pl.ds(nh, nh)], 7, xp)
        r4B.start()
        r4A.wait()
        r4B.wait()

    return pl.pallas_call(
        body,
        out_shape=jax.ShapeDtypeStruct((m, n), jnp.bfloat16),
        in_specs=[pl.BlockSpec(memory_space=pltpu.VMEM)],
        out_specs=pl.BlockSpec(memory_space=pltpu.VMEM),
        scratch_shapes=[
            pltpu.VMEM((half, nh), jnp.bfloat16),
            pltpu.VMEM((half, nh), jnp.bfloat16),
            pltpu.VMEM((half, nh), jnp.bfloat16),
            pltpu.VMEM((half, nh), jnp.bfloat16),
            pltpu.VMEM((half, nh), jnp.bfloat16),
            pltpu.VMEM((half, nh), jnp.bfloat16),
            pltpu.VMEM((quart, nh), jnp.bfloat16),
            pltpu.VMEM((quart, nh), jnp.bfloat16),
            pltpu.SemaphoreType.DMA((8,)),
            pltpu.SemaphoreType.DMA((8,)),
        ],
        compiler_params=pltpu.CompilerParams(collective_id=0),
    )(t)


# device time: 44261 ns/iter; 1.8950x vs baseline; 1.1209x over previous
import jax
import jax.numpy as jnp
from jax import lax
from jax.experimental import pallas as pl
from jax.experimental.pallas import tpu as pltpu

N_DEV = 4
N_CHUNKS = 4


def kernel(t):
    m, n = t.shape
    half = m // 2
    quart = m // 4
    w = n // N_CHUNKS

    def body(t_ref, out_ref, *scratch):
        send1 = scratch[0:N_CHUNKS]
        recv1 = scratch[N_CHUNKS:2 * N_CHUNKS]
        acc = scratch[2 * N_CHUNKS:3 * N_CHUNKS]
        recv2 = scratch[3 * N_CHUNKS:4 * N_CHUNKS]
        send_sems, recv_sems = scratch[4 * N_CHUNKS:]

        my = lax.axis_index("i")
        ybit = (my ^ (my >> 1)) & 1
        xbit = my >> 1
        yp = my ^ 1
        xp = 3 - my

        cfg = []
        for c in range(N_CHUNKS):
            if c % 2 == 0:
                cfg.append((c * w, yp, ybit, xp, xbit))
            else:
                cfg.append((c * w, xp, xbit, yp, ybit))

        def rdma(src, dst, slot, dev):
            return pltpu.make_async_remote_copy(
                src_ref=src, dst_ref=dst,
                send_sem=send_sems.at[slot], recv_sem=recv_sems.at[slot],
                device_id=(dev,), device_id_type=pl.DeviceIdType.MESH,
            )

        def f(s):
            r = jnp.maximum(s, 0.0)
            return jnp.tanh(s) * s * s + r * r * r

        barrier_sem = pltpu.get_barrier_semaphore()
        for nbr in (yp, xp):
            pl.semaphore_signal(
                barrier_sem, inc=1,
                device_id=(nbr,), device_id_type=pl.DeviceIdType.MESH,
            )
        pl.semaphore_wait(barrier_sem, 2)

        r1 = []
        for c, (col0, p1, k1, _, _) in enumerate(cfg):
            send1[c][:, :] = t_ref[
                pl.ds((1 - k1) * half, half), pl.ds(col0, w)
            ].astype(jnp.bfloat16)
            r = rdma(send1[c], recv1[c], c, p1)
            r.start()
            r1.append(r)

        r2 = []
        for c, (col0, p1, k1, p2, k2) in enumerate(cfg):
            r1[c].wait()
            acc[c][:, :] = (
                t_ref[pl.ds(k1 * half, half), pl.ds(col0, w)]
                + recv1[c][:, :].astype(jnp.float32)
            ).astype(jnp.bfloat16)
            r = rdma(acc[c].at[pl.ds((1 - k2) * quart, quart), :],
                     recv2[c], N_CHUNKS + c, p2)
            r.start()
            r2.append(r)

        r3 = []
        for c, (col0, p1, k1, p2, k2) in enumerate(cfg):
            r2[c].wait()
            s = (
                acc[c][pl.ds(k2 * quart, quart), :].astype(jnp.float32)
                + recv2[c][:, :].astype(jnp.float32)
            )
            q0 = k1 * half + k2 * quart
            out_ref[pl.ds(q0, quart), pl.ds(col0, w)] = f(s).astype(jnp.bfloat16)
            r = rdma(out_ref.at[pl.ds(q0, quart), pl.ds(col0, w)],
                     out_ref.at[pl.ds(q0, quart), pl.ds(col0, w)],
                     2 * N_CHUNKS + c, p2)
            r.start()
            r3.append(r)

        r4 = []
        for c, (col0, p1, k1, p2, k2) in enumerate(cfg):
            r3[c].wait()
            r = rdma(out_ref.at[pl.ds(k1 * half, half), pl.ds(col0, w)],
                     out_ref.at[pl.ds(k1 * half, half), pl.ds(col0, w)],
                     3 * N_CHUNKS + c, p1)
            r.start()
            r4.append(r)
        for c in range(N_CHUNKS):
            r4[c].wait()

    return pl.pallas_call(
        body,
        out_shape=jax.ShapeDtypeStruct((m, n), jnp.bfloat16),
        in_specs=[pl.BlockSpec(memory_space=pltpu.VMEM)],
        out_specs=pl.BlockSpec(memory_space=pltpu.VMEM),
        scratch_shapes=(
            [pltpu.VMEM((half, w), jnp.bfloat16)] * N_CHUNKS
            + [pltpu.VMEM((half, w), jnp.bfloat16)] * N_CHUNKS
            + [pltpu.VMEM((half, w), jnp.bfloat16)] * N_CHUNKS
            + [pltpu.VMEM((quart, w), jnp.bfloat16)] * N_CHUNKS
            + [
                pltpu.SemaphoreType.DMA((4 * N_CHUNKS,)),
                pltpu.SemaphoreType.DMA((4 * N_CHUNKS,)),
            ]
        ),
        compiler_params=pltpu.CompilerParams(collective_id=0),
    )(t)
